# speedup vs baseline: 6.0134x; 6.0134x over previous
import math

import jax
import jax.numpy as jnp
from jax.experimental import pallas as pl

_IMG_W = 1333.0
_IMG_H = 800.0
_SCORE_THRESH = 0.05
_NMS_THRESH = 0.5
_DETS = 100
_N = 20000
_C = 81
_CLIP = math.log(1000.0 / 16.0)
_NEG = -1e10

_BN = 1000
_PAD_N = 20480
_ROWS = 160


def _prep_kernel(lg_ref, br_ref, prop_ref,
                 x1_ref, y1_ref, x2_ref, y2_ref, sc_ref, lab_ref):
    lg = lg_ref[...]
    m = jnp.max(lg, axis=1, keepdims=True)
    sumexp = jnp.sum(jnp.exp(lg - m), axis=1, keepdims=True)
    score = 1.0 / sumexp
    cls_iota = jax.lax.broadcasted_iota(jnp.int32, lg.shape, 1)
    label = jnp.min(jnp.where(lg >= m, cls_iota, 2**30), axis=1, keepdims=True)

    prop = prop_ref[...]
    px1 = prop[:, 0:1]
    py1 = prop[:, 1:2]
    px2 = prop[:, 2:3]
    py2 = prop[:, 3:4]
    w = px2 - px1 + 1.0
    h = py2 - py1 + 1.0
    cx = px1 + 0.5 * w
    cy = py1 + 0.5 * h

    br = br_ref[...]
    lane = jax.lax.broadcasted_iota(jnp.int32, br.shape, 1)
    base = label * 4
    dx = jnp.sum(jnp.where(lane == base, br, 0.0), axis=1, keepdims=True) / 10.0
    dy = jnp.sum(jnp.where(lane == base + 1, br, 0.0), axis=1, keepdims=True) / 10.0
    dw = jnp.sum(jnp.where(lane == base + 2, br, 0.0), axis=1, keepdims=True) / 5.0
    dh = jnp.sum(jnp.where(lane == base + 3, br, 0.0), axis=1, keepdims=True) / 5.0
    dw = jnp.minimum(dw, _CLIP)
    dh = jnp.minimum(dh, _CLIP)

    pcx = dx * w + cx
    pcy = dy * h + cy
    pw = jnp.exp(dw) * w
    ph = jnp.exp(dh) * h
    bx1 = jnp.clip(pcx - 0.5 * pw, 0.0, _IMG_W - 1.0)
    by1 = jnp.clip(pcy - 0.5 * ph, 0.0, _IMG_H - 1.0)
    bx2 = jnp.clip(pcx + 0.5 * pw - 1.0, 0.0, _IMG_W - 1.0)
    by2 = jnp.clip(pcy + 0.5 * ph - 1.0, 0.0, _IMG_H - 1.0)

    keep = (label >= 1) & (score > _SCORE_THRESH)
    x1_ref[...] = bx1
    y1_ref[...] = by1
    x2_ref[...] = bx2
    y2_ref[...] = by2
    sc_ref[...] = jnp.where(keep, score, _NEG)
    lab_ref[...] = label


def _nms_kernel(sc_ref, x1_ref, y1_ref, x2_ref, y2_ref, lab_ref,
                ox1_ref, oy1_ref, ox2_ref, oy2_ref, osc_ref, olab_ref):
    x1 = x1_ref[...]
    y1 = y1_ref[...]
    x2 = x2_ref[...]
    y2 = y2_ref[...]
    lab = lab_ref[...]
    area = (x2 - x1 + 1.0) * (y2 - y1 + 1.0)
    lin = (jax.lax.broadcasted_iota(jnp.int32, (_ROWS, 128), 0) * 128
           + jax.lax.broadcasted_iota(jnp.int32, (_ROWS, 128), 1))

    def body(i, scores):
        best = jnp.max(scores)
        bidx = jnp.min(jnp.where(scores == best, lin, 2**30))
        sel = lin == bidx
        bx1 = jnp.max(jnp.where(sel, x1, -1e30))
        by1 = jnp.max(jnp.where(sel, y1, -1e30))
        bx2 = jnp.max(jnp.where(sel, x2, -1e30))
        by2 = jnp.max(jnp.where(sel, y2, -1e30))
        blab = jnp.max(jnp.where(sel, lab, 0))
        barea = (bx2 - bx1 + 1.0) * (by2 - by1 + 1.0)
        xx1 = jnp.maximum(bx1, x1)
        yy1 = jnp.maximum(by1, y1)
        xx2 = jnp.minimum(bx2, x2)
        yy2 = jnp.minimum(by2, y2)
        inter = (jnp.maximum(xx2 - xx1 + 1.0, 0.0)
                 * jnp.maximum(yy2 - yy1 + 1.0, 0.0))
        iou = inter / (barea + area - inter)
        suppress = (iou > _NMS_THRESH) | sel
        new_scores = jnp.where(suppress, _NEG, scores)

        valid = best > 0.0
        zf = jnp.float32(0.0)
        ox1_ref[pl.ds(i, 1), :] = jnp.where(valid, bx1, zf).reshape(1, 1)
        oy1_ref[pl.ds(i, 1), :] = jnp.where(valid, by1, zf).reshape(1, 1)
        ox2_ref[pl.ds(i, 1), :] = jnp.where(valid, bx2, zf).reshape(1, 1)
        oy2_ref[pl.ds(i, 1), :] = jnp.where(valid, by2, zf).reshape(1, 1)
        osc_ref[pl.ds(i, 1), :] = jnp.where(valid, best, zf).reshape(1, 1)
        olab_ref[pl.ds(i, 1), :] = jnp.where(valid, blab, 0).reshape(1, 1)
        return new_scores

    jax.lax.fori_loop(0, _DETS, body, sc_ref[...])


def kernel(class_logits, box_regression, proposal_boxes):
    n_blocks = _N // _BN
    f32 = jnp.float32
    col = jax.ShapeDtypeStruct((_N, 1), f32)
    coli = jax.ShapeDtypeStruct((_N, 1), jnp.int32)
    bspec_in = lambda w: pl.BlockSpec((_BN, w), lambda i: (i, 0))
    bspec_out = pl.BlockSpec((_BN, 1), lambda i: (i, 0))
    x1, y1, x2, y2, sc, lab = pl.pallas_call(
        _prep_kernel,
        grid=(n_blocks,),
        in_specs=[bspec_in(_C), bspec_in(_C * 4), bspec_in(4)],
        out_specs=[bspec_out] * 6,
        out_shape=[col, col, col, col, col, coli],
    )(class_logits, box_regression, proposal_boxes)

    def pad2d(a, fill):
        flat = a[:, 0]
        flat = jnp.pad(flat, (0, _PAD_N - _N), constant_values=fill)
        return flat.reshape(_ROWS, 128)

    sc2 = pad2d(sc, _NEG)
    x12 = pad2d(x1, 0.0)
    y12 = pad2d(y1, 0.0)
    x22 = pad2d(x2, 0.0)
    y22 = pad2d(y2, 0.0)
    lab2 = pad2d(lab, 0)

    ocol = jax.ShapeDtypeStruct((_DETS, 1), f32)
    ocoli = jax.ShapeDtypeStruct((_DETS, 1), jnp.int32)
    ox1, oy1, ox2, oy2, osc, olab = pl.pallas_call(
        _nms_kernel,
        out_shape=[ocol, ocol, ocol, ocol, ocol, ocoli],
    )(sc2, x12, y12, x22, y22, lab2)

    out_boxes = jnp.concatenate([ox1, oy1, ox2, oy2], axis=1)
    return out_boxes, osc[:, 0], olab[:, 0]
